# baseline (speedup 1.0000x reference)
import functools
import jax
import jax.numpy as jnp
from jax import lax
from jax.experimental import pallas as pl
from jax.experimental.pallas import tpu as pltpu
from jax.experimental.pallas import tpu_sc as plsc

BATCH = 16384
EMB = 32
PACK = 4
WIDE = PACK * EMB
NUM_CORES = 2
NUM_SUBCORES = 16
NUM_WORKERS = NUM_CORES * NUM_SUBCORES
BPW = BATCH // NUM_WORKERS
CH = 256
NCH = BPW // CH
IDXV = 128


def _mf_body(ut_hbm, uq_hbm, vt_hbm, vq_hbm, ue_hbm, ie_hbm, out_hbm,
             utile, uquar, vtile, vquar, ubuf, vbuf, outb, sem):
    wid = lax.axis_index("s") * NUM_CORES + lax.axis_index("c")
    base = wid * BPW

    pltpu.sync_copy(ut_hbm.at[wid], utile)
    pltpu.sync_copy(uq_hbm.at[wid], uquar)
    pltpu.sync_copy(vt_hbm.at[wid], vtile)
    pltpu.sync_copy(vq_hbm.at[wid], vquar)

    riota = lax.iota(jnp.int32, 16)

    def chunk(h):
        copies = []
        for p in range(CH // IDXV):
            off = h * CH + p * IDXV
            copies.append(pltpu.async_copy(
                ue_hbm.at[utile.at[pl.ds(off, IDXV)]],
                ubuf.at[pl.ds(p * IDXV, IDXV)], sem))
            copies.append(pltpu.async_copy(
                ie_hbm.at[vtile.at[pl.ds(off, IDXV)]],
                vbuf.at[pl.ds(p * IDXV, IDXV)], sem))
        for cp in copies:
            cp.wait()

        for i in range(CH // 16):
            off = h * CH + i * 16
            jvec = i * 16 + riota
            uq16 = uquar[pl.ds(off, 16)]
            vq16 = vquar[pl.ds(off, 16)]
            acc = jnp.zeros((16,), jnp.float32)
            for c in range(EMB):
                uc = plsc.load_gather(ubuf, [jvec, uq16 + c])
                vc = plsc.load_gather(vbuf, [jvec, vq16 + c])
                acc = acc + uc * vc
            outb[pl.ds(off, 16)] = acc

    for h in range(NCH):
        chunk(h)

    pltpu.sync_copy(outb, out_hbm.at[pl.ds(base, BPW)])


_mf_kernel = functools.partial(
    pl.kernel,
    mesh=plsc.VectorSubcoreMesh(core_axis_name="c", subcore_axis_name="s"),
    out_type=jax.ShapeDtypeStruct((BATCH,), jnp.float32),
    scratch_types=[
        pltpu.VMEM((BPW,), jnp.int32),
        pltpu.VMEM((BPW,), jnp.int32),
        pltpu.VMEM((BPW,), jnp.int32),
        pltpu.VMEM((BPW,), jnp.int32),
        pltpu.VMEM((CH, WIDE), jnp.float32),
        pltpu.VMEM((CH, WIDE), jnp.float32),
        pltpu.VMEM((BPW,), jnp.float32),
        pltpu.SemaphoreType.DMA,
    ],
    compiler_params=pltpu.CompilerParams(needs_layout_passes=False),
)(_mf_body)


@jax.jit
def kernel(u, v, user_emb, item_emb):
    u32 = u.astype(jnp.int32)
    v32 = v.astype(jnp.int32)
    ut = (u32 // PACK).reshape(NUM_WORKERS, BPW)
    uq = ((u32 % PACK) * EMB).reshape(NUM_WORKERS, BPW)
    vt = (v32 // PACK).reshape(NUM_WORKERS, BPW)
    vq = ((v32 % PACK) * EMB).reshape(NUM_WORKERS, BPW)
    ue2 = user_emb.reshape(-1, WIDE)
    ie2 = item_emb.reshape(-1, WIDE)
    return _mf_kernel(ut, uq, vt, vq, ue2, ie2)
